# baseline (speedup 1.0000x reference)
import numpy as np
import jax
import jax.numpy as jnp
from jax import lax
from jax.experimental import pallas as pl

_BB = 32
_NP = 32
_PM = _NP * _NP
_Y0 = 0.28209479177387814
_C1 = 0.4886025119029199
_C2 = 1.0925484305920792


def _sp_core(v):
    return jnp.maximum(v, 0.0) + jnp.log1p(jnp.exp(-jnp.abs(v)))


def _stage1(xyzT_ref, featT_ref, featf_ref, embp_ref, embT_ref, w1_ref,
            w2_ref, w3_ref, wlt_ref, wltT_ref, wr_ref, br_ref, wc_ref,
            bc_ref, out_ref):
    BB = xyzT_ref.shape[0]
    P = BB * _PM
    A = BB * _NP
    f32 = jnp.float32

    qi = lax.broadcasted_iota(jnp.int32, (_NP, _PM), 1)
    ri = lax.broadcasted_iota(jnp.int32, (_NP, _PM), 0)
    repI = ((qi // _NP) == ri).astype(f32)
    repJ = ((qi % _NP) == ri).astype(f32)

    xi_p, xj_p = [], []
    for b in range(BB):
        xyzb = xyzT_ref[b]
        xi_p.append(jnp.dot(xyzb, repI, preferred_element_type=f32,
                            precision=jax.lax.Precision.HIGHEST))
        xj_p.append(jnp.dot(xyzb, repJ, preferred_element_type=f32,
                            precision=jax.lax.Precision.HIGHEST))
    xiT = jnp.concatenate(xi_p, axis=1)
    xjT = jnp.concatenate(xj_p, axis=1)

    rel = xiT - xjT
    d2 = jnp.sum(rel * rel, axis=0, keepdims=True) + 1e-12
    dist = jnp.sqrt(d2)
    u = rel * (1.0 / dist)
    ux, uy, uz = u[0:1], u[1:2], u[2:3]

    lq = lax.broadcasted_iota(jnp.int32, (1, P), 1)
    valid = (((lq // _NP) % _NP) < 30) & ((lq % _NP) < 30)
    mask = ((dist > 1e-6) & (dist < 2.0) & valid).astype(f32)

    krow = lax.broadcasted_iota(jnp.int32, (8, 1), 0)
    diff = dist - krow.astype(f32)
    b8 = jnp.where(jnp.abs(diff) < 1.0, jnp.cos(0.5 * jnp.pi * diff) ** 2, 0.0)
    basisT = jnp.where(krow == 3, 1.0, jnp.where(krow < 3, b8, 0.0))

    ones_row = basisT[3:4]
    h = _sp_core(jnp.dot(w1_ref[...], basisT, preferred_element_type=f32,
                         precision=jax.lax.Precision.HIGHEST))
    h = jnp.concatenate([h, ones_row], axis=0)
    h = _sp_core(jnp.dot(w2_ref[...], h, preferred_element_type=f32))
    h = jnp.concatenate([h, ones_row], axis=0)
    radialT = jnp.dot(w3_ref[...], h, preferred_element_type=f32)

    xtj_p = []
    for b in range(BB):
        ohT = (lax.broadcasted_iota(jnp.int32, (8, _NP), 0).astype(f32)
               == featT_ref[b]).astype(f32)
        xTb = jnp.dot(embT_ref[...], ohT, preferred_element_type=f32)
        xt3Tb = jnp.dot(wltT_ref[...], xTb, preferred_element_type=f32)
        xtj_p.append(jnp.dot(xt3Tb, repJ, preferred_element_type=f32))
    xtjT = jnp.concatenate(xtj_p, axis=1)

    tmpT = radialT * xtjT

    ys = [_Y0 * mask,
          _C1 * uy * mask, _C1 * uz * mask, _C1 * ux * mask,
          _C2 * ux * uy * mask, _C2 * uy * uz * mask,
          0.31539156525252005 * (3.0 * uz * uz - 1.0) * mask,
          _C2 * ux * uz * mask,
          0.5462742152960396 * (ux * ux - uy * uy) * mask]
    ls = (0, 1, 1, 1, 2, 2, 2, 2, 2)

    m_p = []
    for b in range(BB):
        sl = slice(b * _PM, (b + 1) * _PM)
        pieces = [tmpT[ls[k] * 32:(ls[k] + 1) * 32, sl] * ys[k][:, sl]
                  for k in range(9)]
        prod = jnp.concatenate(pieces, axis=0)
        m_p.append(lax.dot_general(repI, prod, (((1,), (1,)), ((), ())),
                                   preferred_element_type=f32))
    M = jnp.concatenate(m_p, axis=0)

    t8 = lax.broadcasted_iota(jnp.int32, (1, 8), 1).astype(f32)
    oh = (featf_ref[...].reshape(A, 1) == t8).astype(f32)
    x = jnp.dot(oh, embp_ref[...], preferred_element_type=f32)

    feats = jnp.concatenate([x, M], axis=1)
    res = feats + jnp.maximum(
        jnp.dot(feats, wr_ref[...], preferred_element_type=f32)
        + br_ref[...], 0.0)

    am = (lax.broadcasted_iota(jnp.int32, (BB, _NP, 1), 1) < 30).astype(f32)
    gmean = jnp.sum(feats.reshape(BB, _NP, 320) * am, axis=1) / 30.0
    rmean = jnp.sum(res.reshape(BB, _NP, 320) * am, axis=1) / 30.0
    g = jnp.concatenate([gmean, rmean], axis=1)
    out_ref[...] = jnp.dot(g, wc_ref[...],
                           preferred_element_type=f32) + bc_ref[...]


def _stage2(h_ref, g_ref, b_ref, wo_ref, bo_ref, out_ref):
    h = h_ref[...]
    mu = jnp.mean(h, axis=0, keepdims=True)
    d = h - mu
    var = jnp.mean(d * d, axis=0, keepdims=True)
    hn = d * lax.rsqrt(var + 1e-5) * g_ref[...] + b_ref[...]
    hl = jnp.where(hn > 0, hn, 0.01 * hn)
    o = jnp.sum(hl * wo_ref[...], axis=1, keepdims=True) + bo_ref[...]
    out_ref[...] = jax.nn.sigmoid(o)


def _lane_perm():
    perm = np.full(320, -1, dtype=np.int64)
    perm[:32] = np.arange(32)
    base = (32, 56, 128)
    for k in range(9):
        l = 0 if k == 0 else (1 if k < 4 else 2)
        m = 0 if k == 0 else (k - 1 if k < 4 else k - 4)
        for c in range(24):
            perm[32 + k * 32 + c] = base[l] + c * (2 * l + 1) + m
    return perm


def kernel(xyz, features, emb_table, rw1, rb1, rw2, rb2, rw3, rb3,
           Wl, Wr, br, Wc, bc, gamma, beta, Wout, bout):
    B, N, _ = xyz.shape
    f32 = jnp.float32

    xyzT = jnp.transpose(jnp.pad(xyz, ((0, 0), (0, _NP - N), (0, 0))),
                         (0, 2, 1))
    featp = jnp.pad(features.astype(f32), ((0, 0), (0, _NP - N)),
                    constant_values=6.0)
    featT = featp[:, None, :]
    featf = featp[..., None]
    embp = jnp.pad(emb_table, ((0, 2), (0, 0)))
    embT = embp.T

    w1 = jnp.zeros((104, 8), f32).at[:100, :3].set(5.0 * rw1.T)
    w1 = w1.at[:100, 3].set(5.0 * rb1)
    w2 = jnp.zeros((104, 105), f32).at[:100, :100].set(rw2.T)
    w2 = w2.at[:100, 104].set(5.0 * rb2)
    w3r = (0.2 * rw3).reshape(100, 3, 24)
    w3 = jnp.zeros((96, 105), f32)
    w3 = w3.at[:, :100].set(
        jnp.pad(jnp.transpose(w3r, (1, 2, 0)), ((0, 0), (0, 8), (0, 0))
                ).reshape(96, 100))
    w3 = w3.at[:, 104].set(
        jnp.pad(rb3.reshape(3, 24), ((0, 0), (0, 8))).reshape(96))
    wlt = jnp.zeros((32, 3, 32), f32).at[:, :, :24].set(
        jnp.transpose(Wl, (2, 0, 1))).reshape(32, 96)
    wltT = wlt.T

    perm = _lane_perm()
    vmask = perm >= 0
    pc = np.where(vmask, perm, 0)
    vm = jnp.asarray(vmask)
    wr = jnp.where(vm[:, None] & vm[None, :], Wr[pc][:, pc], 0.0)
    brp = jnp.where(vm, br[pc], 0.0)[None, :]
    wc = jnp.concatenate([jnp.where(vm[:, None], Wc[pc, :], 0.0),
                          jnp.where(vm[:, None], Wc[pc + 248, :], 0.0)],
                         axis=0)
    bc2 = bc[None, :]

    full = lambda a: pl.BlockSpec(a.shape, lambda i: (0,) * a.ndim)
    weights = (embp, embT, w1, w2, w3, wlt, wltT, wr, brp, wc, bc2)
    hcol = pl.pallas_call(
        _stage1,
        grid=(B // _BB,),
        in_specs=[
            pl.BlockSpec((_BB, 3, _NP), lambda i: (i, 0, 0)),
            pl.BlockSpec((_BB, 1, _NP), lambda i: (i, 0, 0)),
            pl.BlockSpec((_BB, _NP, 1), lambda i: (i, 0, 0)),
        ] + [full(a) for a in weights],
        out_specs=pl.BlockSpec((_BB, 128), lambda i: (i, 0)),
        out_shape=jax.ShapeDtypeStruct((B, 128), f32),
    )(xyzT, featT, featf, *weights)

    out = pl.pallas_call(
        _stage2,
        out_shape=jax.ShapeDtypeStruct((B, 1), f32),
    )(hcol, gamma[None, :], beta[None, :], Wout.T, bout[None, :])
    return out


# speedup vs baseline: 1.0283x; 1.0283x over previous
import numpy as np
import jax
import jax.numpy as jnp
from jax import lax
from jax.experimental import pallas as pl

_BB = 16
_NP = 32
_PM = _NP * _NP
_Y0 = 0.28209479177387814
_C1 = 0.4886025119029199
_C2 = 1.0925484305920792


def _sp_core(v):
    return jnp.maximum(v, 0.0) + jnp.log1p(jnp.exp(-jnp.abs(v)))


def _stage1(xyzT_ref, featT_ref, featf_ref, embp_ref, embT_ref, w1_ref,
            w2_ref, w3_ref, wlt_ref, wltT_ref, wr_ref, br_ref, wc_ref,
            bc_ref, out_ref):
    BB = xyzT_ref.shape[0]
    P = BB * _PM
    A = BB * _NP
    f32 = jnp.float32

    qi = lax.broadcasted_iota(jnp.int32, (_NP, _PM), 1)
    ri = lax.broadcasted_iota(jnp.int32, (_NP, _PM), 0)
    repI = ((qi // _NP) == ri).astype(f32)
    repJ = ((qi % _NP) == ri).astype(f32)

    xi_p, xj_p = [], []
    for b in range(BB):
        xyzb = xyzT_ref[b]
        xi_p.append(jnp.dot(xyzb, repI, preferred_element_type=f32,
                            precision=jax.lax.Precision.HIGHEST))
        xj_p.append(jnp.dot(xyzb, repJ, preferred_element_type=f32,
                            precision=jax.lax.Precision.HIGHEST))
    xiT = jnp.concatenate(xi_p, axis=1)
    xjT = jnp.concatenate(xj_p, axis=1)

    rel = xiT - xjT
    d2 = jnp.sum(rel * rel, axis=0, keepdims=True) + 1e-12
    dist = jnp.sqrt(d2)
    u = rel * (1.0 / dist)
    ux, uy, uz = u[0:1], u[1:2], u[2:3]

    lq = lax.broadcasted_iota(jnp.int32, (1, P), 1)
    valid = (((lq // _NP) % _NP) < 30) & ((lq % _NP) < 30)
    mask = ((dist > 1e-6) & (dist < 2.0) & valid).astype(f32)

    krow = lax.broadcasted_iota(jnp.int32, (8, 1), 0)
    diff = dist - krow.astype(f32)
    b8 = jnp.where(jnp.abs(diff) < 1.0, jnp.cos(0.5 * jnp.pi * diff) ** 2, 0.0)
    basisT = jnp.where(krow == 3, 1.0, b8)

    h = _sp_core(jnp.dot(w1_ref[...], basisT, preferred_element_type=f32,
                         precision=jax.lax.Precision.HIGHEST))
    h = _sp_core(jnp.dot(w2_ref[...], h, preferred_element_type=f32))
    radialT = jnp.dot(w3_ref[...], h, preferred_element_type=f32)

    xtj_p = []
    for b in range(BB):
        ohT = (lax.broadcasted_iota(jnp.int32, (8, _NP), 0).astype(f32)
               == featT_ref[b]).astype(f32)
        xTb = jnp.dot(embT_ref[...], ohT, preferred_element_type=f32)
        xt3Tb = jnp.dot(wltT_ref[...], xTb, preferred_element_type=f32)
        xtj_p.append(jnp.dot(xt3Tb, repJ, preferred_element_type=f32))
    xtjT = jnp.concatenate(xtj_p, axis=1)

    tmpT = radialT * xtjT

    ys = [_Y0 * mask,
          _C1 * uy * mask, _C1 * uz * mask, _C1 * ux * mask,
          _C2 * ux * uy * mask, _C2 * uy * uz * mask,
          0.31539156525252005 * (3.0 * uz * uz - 1.0) * mask,
          _C2 * ux * uz * mask,
          0.5462742152960396 * (ux * ux - uy * uy) * mask]
    ls = (0, 1, 1, 1, 2, 2, 2, 2, 2)

    m_p = []
    for b in range(BB):
        sl = slice(b * _PM, (b + 1) * _PM)
        pieces = [tmpT[ls[k] * 32:(ls[k] + 1) * 32, sl] * ys[k][:, sl]
                  for k in range(9)]
        prod = jnp.concatenate(pieces, axis=0)
        m_p.append(lax.dot_general(repI, prod, (((1,), (1,)), ((), ())),
                                   preferred_element_type=f32))
    M = jnp.concatenate(m_p, axis=0)

    t8 = lax.broadcasted_iota(jnp.int32, (1, 8), 1).astype(f32)
    oh = (featf_ref[...].reshape(A, 1) == t8).astype(f32)
    x = jnp.dot(oh, embp_ref[...], preferred_element_type=f32)

    feats = jnp.concatenate([x, M], axis=1)
    res = feats + jnp.maximum(
        jnp.dot(feats, wr_ref[...], preferred_element_type=f32)
        + br_ref[...], 0.0)

    am = (lax.broadcasted_iota(jnp.int32, (BB, _NP, 1), 1) < 30).astype(f32)
    gmean = jnp.sum(feats.reshape(BB, _NP, 320) * am, axis=1) / 30.0
    rmean = jnp.sum(res.reshape(BB, _NP, 320) * am, axis=1) / 30.0
    g = jnp.concatenate([gmean, rmean], axis=1)
    out_ref[...] = jnp.dot(g, wc_ref[...],
                           preferred_element_type=f32) + bc_ref[...]


def _stage2(h_ref, g_ref, b_ref, wo_ref, bo_ref, out_ref):
    h = h_ref[...]
    mu = jnp.mean(h, axis=0, keepdims=True)
    d = h - mu
    var = jnp.mean(d * d, axis=0, keepdims=True)
    hn = d * lax.rsqrt(var + 1e-5) * g_ref[...] + b_ref[...]
    hl = jnp.where(hn > 0, hn, 0.01 * hn)
    o = jnp.sum(hl * wo_ref[...], axis=1, keepdims=True) + bo_ref[...]
    out_ref[...] = jax.nn.sigmoid(o)


def _lane_perm():
    perm = np.full(320, -1, dtype=np.int64)
    perm[:32] = np.arange(32)
    base = (32, 56, 128)
    for k in range(9):
        l = 0 if k == 0 else (1 if k < 4 else 2)
        m = 0 if k == 0 else (k - 1 if k < 4 else k - 4)
        for c in range(24):
            perm[32 + k * 32 + c] = base[l] + c * (2 * l + 1) + m
    return perm


def kernel(xyz, features, emb_table, rw1, rb1, rw2, rb2, rw3, rb3,
           Wl, Wr, br, Wc, bc, gamma, beta, Wout, bout):
    B, N, _ = xyz.shape
    f32 = jnp.float32

    xyzT = jnp.transpose(jnp.pad(xyz, ((0, 0), (0, _NP - N), (0, 0))),
                         (0, 2, 1))
    featp = jnp.pad(features.astype(f32), ((0, 0), (0, _NP - N)),
                    constant_values=6.0)
    featT = featp[:, None, :]
    featf = featp[..., None]
    embp = jnp.pad(emb_table, ((0, 2), (0, 0)))
    embT = embp.T

    w1 = jnp.zeros((104, 8), f32).at[:100, :3].set(5.0 * rw1.T)
    w1 = w1.at[:100, 3].set(5.0 * rb1)
    ln2 = np.float32(np.log(2.0))
    w2 = jnp.zeros((104, 104), f32).at[:100, :100].set(rw2.T)
    w2 = w2.at[:100, 100:104].set(
        jnp.broadcast_to((5.0 * rb2 / (4.0 * ln2))[:, None], (100, 4)))
    w3r = (0.2 * rw3).reshape(100, 3, 24)
    w3 = jnp.zeros((96, 104), f32)
    w3 = w3.at[:, :100].set(
        jnp.pad(jnp.transpose(w3r, (1, 2, 0)), ((0, 0), (0, 8), (0, 0))
                ).reshape(96, 100))
    w3 = w3.at[:, 100:104].set(jnp.broadcast_to(
        (jnp.pad(rb3.reshape(3, 24), ((0, 0), (0, 8))).reshape(96)
         / (4.0 * ln2))[:, None], (96, 4)))
    wlt = jnp.zeros((32, 3, 32), f32).at[:, :, :24].set(
        jnp.transpose(Wl, (2, 0, 1))).reshape(32, 96)
    wltT = wlt.T

    perm = _lane_perm()
    vmask = perm >= 0
    pc = np.where(vmask, perm, 0)
    vm = jnp.asarray(vmask)
    wr = jnp.where(vm[:, None] & vm[None, :], Wr[pc][:, pc], 0.0)
    brp = jnp.where(vm, br[pc], 0.0)[None, :]
    wc = jnp.concatenate([jnp.where(vm[:, None], Wc[pc, :], 0.0),
                          jnp.where(vm[:, None], Wc[pc + 248, :], 0.0)],
                         axis=0)
    bc2 = bc[None, :]

    full = lambda a: pl.BlockSpec(a.shape, lambda i: (0,) * a.ndim)
    weights = (embp, embT, w1, w2, w3, wlt, wltT, wr, brp, wc, bc2)
    hcol = pl.pallas_call(
        _stage1,
        grid=(B // _BB,),
        in_specs=[
            pl.BlockSpec((_BB, 3, _NP), lambda i: (i, 0, 0)),
            pl.BlockSpec((_BB, 1, _NP), lambda i: (i, 0, 0)),
            pl.BlockSpec((_BB, _NP, 1), lambda i: (i, 0, 0)),
        ] + [full(a) for a in weights],
        out_specs=pl.BlockSpec((_BB, 128), lambda i: (i, 0)),
        out_shape=jax.ShapeDtypeStruct((B, 128), f32),
    )(xyzT, featT, featf, *weights)

    out = pl.pallas_call(
        _stage2,
        out_shape=jax.ShapeDtypeStruct((B, 1), f32),
    )(hcol, gamma[None, :], beta[None, :], Wout.T, bout[None, :])
    return out


# speedup vs baseline: 1.1220x; 1.0912x over previous
import numpy as np
import jax
import jax.numpy as jnp
from jax import lax
from jax.experimental import pallas as pl

_BB = 16
_NP = 32
_PM = _NP * _NP
_Y0 = 0.28209479177387814
_C1 = 0.4886025119029199
_C2 = 1.0925484305920792


def _sp_core(v):
    return jnp.maximum(v, 0.0) + jnp.log2(1.0 + jnp.exp2(-jnp.abs(v)))


def _stage1(xyzT_ref, featT_ref, featf_ref, embp_ref, embT_ref, w1_ref,
            w2_ref, w3_ref, wlt_ref, wltT_ref, wr_ref, br_ref, wc_ref,
            bc_ref, out_ref):
    BB = xyzT_ref.shape[0]
    P = BB * _PM
    A = BB * _NP
    f32 = jnp.float32

    qi = lax.broadcasted_iota(jnp.int32, (_NP, _PM), 1)
    ri = lax.broadcasted_iota(jnp.int32, (_NP, _PM), 0)
    repI = ((qi // _NP) == ri).astype(f32)
    repJ = ((qi % _NP) == ri).astype(f32)

    relD = repI - repJ
    rel_p = []
    for b in range(BB):
        rel_p.append(jnp.dot(xyzT_ref[b], relD, preferred_element_type=f32,
                             precision=jax.lax.Precision.HIGHEST))
    rel = jnp.concatenate(rel_p, axis=1)
    d2 = jnp.sum(rel * rel, axis=0, keepdims=True) + 1e-12
    dist = jnp.sqrt(d2)
    u = rel * (1.0 / dist)
    ux, uy, uz = u[0:1], u[1:2], u[2:3]

    lq = lax.broadcasted_iota(jnp.int32, (1, P), 1)
    valid = (((lq // _NP) % _NP) < 30) & ((lq % _NP) < 30)
    mask = ((dist > 1e-6) & (dist < 2.0) & valid).astype(f32)

    krow = lax.broadcasted_iota(jnp.int32, (8, 1), 0)
    diff = dist - krow.astype(f32)
    b8 = jnp.where(jnp.abs(diff) < 1.0, jnp.cos(0.5 * jnp.pi * diff) ** 2, 0.0)
    basisT = jnp.where(krow == 3, 1.0, b8)

    h = _sp_core(jnp.dot(w1_ref[...], basisT, preferred_element_type=f32,
                         precision=jax.lax.Precision.HIGHEST))
    h = _sp_core(jnp.dot(w2_ref[...], h, preferred_element_type=f32))
    radialT = jnp.dot(w3_ref[...], h, preferred_element_type=f32)

    xtj_p = []
    for b in range(BB):
        ohT = (lax.broadcasted_iota(jnp.int32, (8, _NP), 0).astype(f32)
               == featT_ref[b]).astype(f32)
        xTb = jnp.dot(embT_ref[...], ohT, preferred_element_type=f32)
        xt3Tb = jnp.dot(wltT_ref[...], xTb, preferred_element_type=f32)
        xtj_p.append(jnp.dot(xt3Tb, repJ, preferred_element_type=f32))
    xtjT = jnp.concatenate(xtj_p, axis=1)

    tmpT = radialT * xtjT

    ys = [_Y0 * mask,
          _C1 * uy * mask, _C1 * uz * mask, _C1 * ux * mask,
          _C2 * ux * uy * mask, _C2 * uy * uz * mask,
          0.31539156525252005 * (3.0 * uz * uz - 1.0) * mask,
          _C2 * ux * uz * mask,
          0.5462742152960396 * (ux * ux - uy * uy) * mask]
    ls = (0, 1, 1, 1, 2, 2, 2, 2, 2)

    m_p = []
    for b in range(BB):
        sl = slice(b * _PM, (b + 1) * _PM)
        pieces = [tmpT[ls[k] * 32:(ls[k] + 1) * 32, sl] * ys[k][:, sl]
                  for k in range(9)]
        prod = jnp.concatenate(pieces, axis=0)
        m_p.append(lax.dot_general(repI, prod, (((1,), (1,)), ((), ())),
                                   preferred_element_type=f32))
    M = jnp.concatenate(m_p, axis=0)

    t8 = lax.broadcasted_iota(jnp.int32, (1, 8), 1).astype(f32)
    oh = (featf_ref[...].reshape(A, 1) == t8).astype(f32)
    x = jnp.dot(oh, embp_ref[...], preferred_element_type=f32)

    feats = jnp.concatenate([x, M], axis=1)
    res = feats + jnp.maximum(
        jnp.dot(feats, wr_ref[...], preferred_element_type=f32)
        + br_ref[...], 0.0)

    am = (lax.broadcasted_iota(jnp.int32, (BB, _NP, 1), 1) < 30).astype(f32)
    gmean = jnp.sum(feats.reshape(BB, _NP, 320) * am, axis=1) / 30.0
    rmean = jnp.sum(res.reshape(BB, _NP, 320) * am, axis=1) / 30.0
    g = jnp.concatenate([gmean, rmean], axis=1)
    out_ref[...] = jnp.dot(g, wc_ref[...],
                           preferred_element_type=f32) + bc_ref[...]


def _stage2(h_ref, g_ref, b_ref, wo_ref, bo_ref, out_ref):
    h = h_ref[...]
    mu = jnp.mean(h, axis=0, keepdims=True)
    d = h - mu
    var = jnp.mean(d * d, axis=0, keepdims=True)
    hn = d * lax.rsqrt(var + 1e-5) * g_ref[...] + b_ref[...]
    hl = jnp.where(hn > 0, hn, 0.01 * hn)
    o = jnp.sum(hl * wo_ref[...], axis=1, keepdims=True) + bo_ref[...]
    out_ref[...] = jax.nn.sigmoid(o)


def _lane_perm():
    perm = np.full(320, -1, dtype=np.int64)
    perm[:32] = np.arange(32)
    base = (32, 56, 128)
    for k in range(9):
        l = 0 if k == 0 else (1 if k < 4 else 2)
        m = 0 if k == 0 else (k - 1 if k < 4 else k - 4)
        for c in range(24):
            perm[32 + k * 32 + c] = base[l] + c * (2 * l + 1) + m
    return perm


def kernel(xyz, features, emb_table, rw1, rb1, rw2, rb2, rw3, rb3,
           Wl, Wr, br, Wc, bc, gamma, beta, Wout, bout):
    B, N, _ = xyz.shape
    f32 = jnp.float32

    xyzT = jnp.transpose(jnp.pad(xyz, ((0, 0), (0, _NP - N), (0, 0))),
                         (0, 2, 1))
    featp = jnp.pad(features.astype(f32), ((0, 0), (0, _NP - N)),
                    constant_values=6.0)
    featT = featp[:, None, :]
    featf = featp[..., None]
    embp = jnp.pad(emb_table, ((0, 2), (0, 0)))
    embT = embp.T

    log2e = np.float32(1.0 / np.log(2.0))
    ln2 = np.float32(np.log(2.0))
    w1 = jnp.zeros((104, 8), f32).at[:100, :3].set(log2e * 5.0 * rw1.T)
    w1 = w1.at[:100, 3].set(log2e * 5.0 * rb1)
    w2 = jnp.zeros((104, 104), f32).at[:100, :100].set(rw2.T)
    w2 = w2.at[:100, 100:104].set(
        jnp.broadcast_to((log2e * 5.0 * rb2 / 4.0)[:, None], (100, 4)))
    w3r = (ln2 * 0.2 * rw3).reshape(100, 3, 24)
    w3 = jnp.zeros((96, 104), f32)
    w3 = w3.at[:, :100].set(
        jnp.pad(jnp.transpose(w3r, (1, 2, 0)), ((0, 0), (0, 8), (0, 0))
                ).reshape(96, 100))
    w3 = w3.at[:, 100:104].set(jnp.broadcast_to(
        (jnp.pad(rb3.reshape(3, 24), ((0, 0), (0, 8))).reshape(96)
         / 4.0)[:, None], (96, 4)))
    wlt = jnp.zeros((32, 3, 32), f32).at[:, :, :24].set(
        jnp.transpose(Wl, (2, 0, 1))).reshape(32, 96)
    wltT = wlt.T

    perm = _lane_perm()
    vmask = perm >= 0
    pc = np.where(vmask, perm, 0)
    vm = jnp.asarray(vmask)
    wr = jnp.where(vm[:, None] & vm[None, :], Wr[pc][:, pc], 0.0)
    brp = jnp.where(vm, br[pc], 0.0)[None, :]
    wc = jnp.concatenate([jnp.where(vm[:, None], Wc[pc, :], 0.0),
                          jnp.where(vm[:, None], Wc[pc + 248, :], 0.0)],
                         axis=0)
    bc2 = bc[None, :]

    full = lambda a: pl.BlockSpec(a.shape, lambda i: (0,) * a.ndim)
    weights = (embp, embT, w1, w2, w3, wlt, wltT, wr, brp, wc, bc2)
    hcol = pl.pallas_call(
        _stage1,
        grid=(B // _BB,),
        in_specs=[
            pl.BlockSpec((_BB, 3, _NP), lambda i: (i, 0, 0)),
            pl.BlockSpec((_BB, 1, _NP), lambda i: (i, 0, 0)),
            pl.BlockSpec((_BB, _NP, 1), lambda i: (i, 0, 0)),
        ] + [full(a) for a in weights],
        out_specs=pl.BlockSpec((_BB, 128), lambda i: (i, 0)),
        out_shape=jax.ShapeDtypeStruct((B, 128), f32),
    )(xyzT, featT, featf, *weights)

    out = pl.pallas_call(
        _stage2,
        out_shape=jax.ShapeDtypeStruct((B, 1), f32),
    )(hcol, gamma[None, :], beta[None, :], Wout.T, bout[None, :])
    return out


# speedup vs baseline: 1.2718x; 1.1335x over previous
import numpy as np
import jax
import jax.numpy as jnp
from jax import lax
from jax.experimental import pallas as pl

_BB = 16
_NP = 32
_PM = _NP * _NP
_Y0 = 0.28209479177387814
_C1 = 0.4886025119029199
_C2 = 1.0925484305920792


def _sp_core(v):
    return jnp.maximum(v, 0.0) + jnp.log2(1.0 + jnp.exp2(-jnp.abs(v)))


def _stage1(xyzT_ref, featT_ref, featf_ref, embp_ref, embT_ref, w1_ref,
            w2_ref, w3_ref, wlt_ref, wltT_ref, wr_ref, br_ref, wc_ref,
            bc_ref, out_ref):
    BB = xyzT_ref.shape[0]
    P = BB * _PM
    A = BB * _NP
    f32 = jnp.float32

    qi = lax.broadcasted_iota(jnp.int32, (_NP, _PM), 1)
    ri = lax.broadcasted_iota(jnp.int32, (_NP, _PM), 0)
    repI = ((qi // _NP) == ri).astype(f32)
    repJ = ((qi % _NP) == ri).astype(f32)

    relD = repI - repJ
    rel_p = []
    for b in range(BB):
        rel_p.append(jnp.dot(xyzT_ref[b], relD, preferred_element_type=f32,
                             precision=jax.lax.Precision.HIGHEST))
    rel = jnp.concatenate(rel_p, axis=1)
    d2 = jnp.sum(rel * rel, axis=0, keepdims=True) + 1e-12
    dist = jnp.sqrt(d2)
    u = rel * (1.0 / dist)
    ux, uy, uz = u[0:1], u[1:2], u[2:3]

    lq = lax.broadcasted_iota(jnp.int32, (1, P), 1)
    valid = (((lq // _NP) % _NP) < 30) & ((lq % _NP) < 30)
    mask = ((dist > 1e-6) & (dist < 2.0) & valid).astype(f32)

    krow = lax.broadcasted_iota(jnp.int32, (8, 1), 0)
    diff = dist - krow.astype(f32)
    b8 = jnp.where(jnp.abs(diff) < 1.0, jnp.cos(0.5 * jnp.pi * diff) ** 2, 0.0)
    basisT = jnp.where(krow == 3, 1.0, b8)

    h = _sp_core(jnp.dot(w1_ref[...], basisT, preferred_element_type=f32))
    h = _sp_core(jnp.dot(w2_ref[...], h, preferred_element_type=f32))
    radialT = jnp.dot(w3_ref[...], h, preferred_element_type=f32)

    xtj_p = []
    for b in range(BB):
        ohT = (lax.broadcasted_iota(jnp.int32, (8, _NP), 0).astype(f32)
               == featT_ref[b]).astype(f32)
        xTb = jnp.dot(embT_ref[...], ohT, preferred_element_type=f32)
        xt3Tb = jnp.dot(wltT_ref[...], xTb, preferred_element_type=f32)
        xtj_p.append(jnp.dot(xt3Tb, repJ, preferred_element_type=f32))
    xtjT = jnp.concatenate(xtj_p, axis=1)

    tmpT = radialT * xtjT

    ys = [_Y0 * mask,
          _C1 * uy * mask, _C1 * uz * mask, _C1 * ux * mask,
          _C2 * ux * uy * mask, _C2 * uy * uz * mask,
          0.31539156525252005 * (3.0 * uz * uz - 1.0) * mask,
          _C2 * ux * uz * mask,
          0.5462742152960396 * (ux * ux - uy * uy) * mask]
    ls = (0, 1, 1, 1, 2, 2, 2, 2, 2)

    m_p = []
    for b in range(BB):
        sl = slice(b * _PM, (b + 1) * _PM)
        pieces = [tmpT[ls[k] * 32:(ls[k] + 1) * 32, sl] * ys[k][:, sl]
                  for k in range(9)]
        prod = jnp.concatenate(pieces, axis=0)
        m_p.append(lax.dot_general(repI, prod, (((1,), (1,)), ((), ())),
                                   preferred_element_type=f32))
    M = jnp.concatenate(m_p, axis=0)

    t8 = lax.broadcasted_iota(jnp.int32, (1, 8), 1).astype(f32)
    oh = (featf_ref[...].reshape(A, 1) == t8).astype(f32)
    x = jnp.dot(oh, embp_ref[...], preferred_element_type=f32)

    feats = jnp.concatenate([x, M], axis=1)
    res = feats + jnp.maximum(
        jnp.dot(feats, wr_ref[...], preferred_element_type=f32)
        + br_ref[...], 0.0)

    am = (lax.broadcasted_iota(jnp.int32, (BB, _NP, 1), 1) < 30).astype(f32)
    gmean = jnp.sum(feats.reshape(BB, _NP, 320) * am, axis=1) / 30.0
    rmean = jnp.sum(res.reshape(BB, _NP, 320) * am, axis=1) / 30.0
    g = jnp.concatenate([gmean, rmean], axis=1)
    out_ref[...] = jnp.dot(g, wc_ref[...],
                           preferred_element_type=f32) + bc_ref[...]


def _stage2(h_ref, g_ref, b_ref, wo_ref, bo_ref, out_ref):
    h = h_ref[...]
    mu = jnp.mean(h, axis=0, keepdims=True)
    d = h - mu
    var = jnp.mean(d * d, axis=0, keepdims=True)
    hn = d * lax.rsqrt(var + 1e-5) * g_ref[...] + b_ref[...]
    hl = jnp.where(hn > 0, hn, 0.01 * hn)
    o = jnp.sum(hl * wo_ref[...], axis=1, keepdims=True) + bo_ref[...]
    out_ref[...] = jax.nn.sigmoid(o)


def _lane_perm():
    perm = np.full(320, -1, dtype=np.int64)
    perm[:32] = np.arange(32)
    base = (32, 56, 128)
    for k in range(9):
        l = 0 if k == 0 else (1 if k < 4 else 2)
        m = 0 if k == 0 else (k - 1 if k < 4 else k - 4)
        for c in range(24):
            perm[32 + k * 32 + c] = base[l] + c * (2 * l + 1) + m
    return perm


def kernel(xyz, features, emb_table, rw1, rb1, rw2, rb2, rw3, rb3,
           Wl, Wr, br, Wc, bc, gamma, beta, Wout, bout):
    B, N, _ = xyz.shape
    f32 = jnp.float32

    xyzT = jnp.transpose(jnp.pad(xyz, ((0, 0), (0, _NP - N), (0, 0))),
                         (0, 2, 1))
    featp = jnp.pad(features.astype(f32), ((0, 0), (0, _NP - N)),
                    constant_values=6.0)
    featT = featp[:, None, :]
    featf = featp[..., None]
    embp = jnp.pad(emb_table, ((0, 2), (0, 0)))
    embT = embp.T

    log2e = np.float32(1.0 / np.log(2.0))
    ln2 = np.float32(np.log(2.0))
    w1 = jnp.zeros((104, 8), f32).at[:100, :3].set(log2e * 5.0 * rw1.T)
    w1 = w1.at[:100, 3].set(log2e * 5.0 * rb1)
    w2 = jnp.zeros((104, 104), f32).at[:100, :100].set(rw2.T)
    w2 = w2.at[:100, 100:104].set(
        jnp.broadcast_to((log2e * 5.0 * rb2 / 4.0)[:, None], (100, 4)))
    w3r = (ln2 * 0.2 * rw3).reshape(100, 3, 24)
    w3 = jnp.zeros((96, 104), f32)
    w3 = w3.at[:, :100].set(
        jnp.pad(jnp.transpose(w3r, (1, 2, 0)), ((0, 0), (0, 8), (0, 0))
                ).reshape(96, 100))
    w3 = w3.at[:, 100:104].set(jnp.broadcast_to(
        (jnp.pad(rb3.reshape(3, 24), ((0, 0), (0, 8))).reshape(96)
         / 4.0)[:, None], (96, 4)))
    wlt = jnp.zeros((32, 3, 32), f32).at[:, :, :24].set(
        jnp.transpose(Wl, (2, 0, 1))).reshape(32, 96)
    wltT = wlt.T

    perm = _lane_perm()
    vmask = perm >= 0
    pc = np.where(vmask, perm, 0)
    vm = jnp.asarray(vmask)
    wr = jnp.where(vm[:, None] & vm[None, :], Wr[pc][:, pc], 0.0)
    brp = jnp.where(vm, br[pc], 0.0)[None, :]
    wc = jnp.concatenate([jnp.where(vm[:, None], Wc[pc, :], 0.0),
                          jnp.where(vm[:, None], Wc[pc + 248, :], 0.0)],
                         axis=0)
    bc2 = bc[None, :]

    full = lambda a: pl.BlockSpec(a.shape, lambda i: (0,) * a.ndim)
    weights = (embp, embT, w1, w2, w3, wlt, wltT, wr, brp, wc, bc2)
    hcol = pl.pallas_call(
        _stage1,
        grid=(B // _BB,),
        in_specs=[
            pl.BlockSpec((_BB, 3, _NP), lambda i: (i, 0, 0)),
            pl.BlockSpec((_BB, 1, _NP), lambda i: (i, 0, 0)),
            pl.BlockSpec((_BB, _NP, 1), lambda i: (i, 0, 0)),
        ] + [full(a) for a in weights],
        out_specs=pl.BlockSpec((_BB, 128), lambda i: (i, 0)),
        out_shape=jax.ShapeDtypeStruct((B, 128), f32),
    )(xyzT, featT, featf, *weights)

    out = pl.pallas_call(
        _stage2,
        out_shape=jax.ShapeDtypeStruct((B, 1), f32),
    )(hcol, gamma[None, :], beta[None, :], Wout.T, bout[None, :])
    return out
